# baseline (speedup 1.0000x reference)
import functools

import jax
import jax.numpy as jnp
from jax import lax
from jax.experimental import pallas as pl
from jax.experimental.pallas import tpu as pltpu
from jax.experimental.pallas import tpu_sc as plsc

BATCH = 16384
EMBED_DIM = 64

_NC = 2
_NS = 16
_NW = _NC * _NS
_BPW = BATCH // _NW
_G = 16
_NGRP = _BPW // _G


def _gather_kernel(idx_hbm, table_hbm, out_hbm, idx_v, buf_v, out_v,
                   sem_a, sem_b):
    wid = lax.axis_index("s") * _NC + lax.axis_index("c")
    base = wid * _BPW
    iota = lax.iota(jnp.int32, 16)

    pltpu.sync_copy(idx_hbm.at[wid], idx_v)

    def chunk_of(g):
        return idx_v[g >> 3, pl.ds((g & 7) * 16, 16)]

    def scalar_at(vec, i):
        return jnp.max(jnp.where(iota == i, vec, 0))

    def issue_group(g, p, sem):
        chunk = chunk_of(g)
        for i in range(_G):
            off = pl.multiple_of((scalar_at(chunk, i) >> 3) * 8, 8)
            pltpu.async_copy(
                table_hbm.at[pl.ds(off, 8)], buf_v.at[p, i], sem)

    def drain_group(p, sem):
        for i in range(_G):
            pltpu.make_async_copy(
                table_hbm.at[pl.ds(0, 8)], buf_v.at[p, i], sem).wait()

    def extract_group(g, p):
        j_vec = jnp.full((16,), g >> 3, jnp.int32)
        for i in range(_G):
            r = g * _G + i
            col = jnp.full((16,), (g & 7) * 16 + i, jnp.int32)
            s_vec = plsc.load_gather(idx_v, [j_vec, col]) & 7
            for k in range(EMBED_DIM // 16):
                val = plsc.load_gather(
                    buf_v.at[p, i], [s_vec, iota + 16 * k])
                out_v[r, pl.ds(16 * k, 16)] = val

    issue_group(0, 0, sem_a)
    issue_group(1, 1, sem_b)

    def body(gp, carry):
        g0 = gp * 2
        drain_group(0, sem_a)
        extract_group(g0, 0)

        @pl.when(g0 + 2 < _NGRP)
        def _():
            issue_group(g0 + 2, 0, sem_a)

        drain_group(1, sem_b)
        extract_group(g0 + 1, 1)

        @pl.when(g0 + 3 < _NGRP)
        def _():
            issue_group(g0 + 3, 1, sem_b)

        return carry

    lax.fori_loop(0, _NGRP // 2, body, 0)

    pltpu.sync_copy(out_v, out_hbm.at[pl.ds(base, _BPW)])


@jax.jit
def _embed_gather(idx_r, table):
    mesh = plsc.VectorSubcoreMesh(core_axis_name="c", subcore_axis_name="s")
    run = functools.partial(
        pl.kernel,
        mesh=mesh,
        out_type=jax.ShapeDtypeStruct((BATCH, EMBED_DIM), jnp.float32),
        scratch_types=[
            pltpu.VMEM((_BPW // 128, 128), jnp.int32),
            pltpu.VMEM((2, _G, 8, EMBED_DIM), jnp.float32),
            pltpu.VMEM((_BPW, EMBED_DIM), jnp.float32),
            pltpu.SemaphoreType.DMA,
            pltpu.SemaphoreType.DMA,
        ],
        compiler_params=pltpu.CompilerParams(needs_layout_passes=False),
    )(_gather_kernel)
    return run(idx_r, table)


_VOCAB = 1000000
_TALIGNED = 999936
_NTSLAB = _TALIGNED // 128


def _transpose_kernel(table_t_hbm, tail_hbm, out_hbm, in_v, tr_v, tail_v):
    wid = lax.axis_index("s") * _NC + lax.axis_index("c")
    iota = lax.iota(jnp.int32, 16)
    nper = (_NTSLAB + _NW - 1) // _NW

    def tpose_block(src, nrows, v_base):
        def vbody(v4, carry):
            for u in range(4):
                v = v4 * 4 + u
                f = jnp.full((16,), v, jnp.int32)
                for k in range(EMBED_DIM // 16):
                    val = plsc.load_gather(src, [iota + 16 * k, f])
                    tr_v[v, pl.ds(16 * k, 16)] = val
            return carry

        lax.fori_loop(0, nrows // 4, vbody, 0)
        pltpu.sync_copy(tr_v.at[pl.ds(0, nrows)],
                        out_hbm.at[pl.ds(v_base, nrows)])

    def body(t, carry):
        sl = t * _NW + wid

        @pl.when(sl < _NTSLAB)
        def _():
            v0 = pl.multiple_of(sl * 128, 128)
            pltpu.sync_copy(table_t_hbm.at[:, pl.ds(v0, 128)], in_v)
            tpose_block(in_v, 128, v0)

        return carry

    lax.fori_loop(0, nper, body, 0)

    @pl.when(wid == 0)
    def _():
        pltpu.sync_copy(tail_hbm, tail_v)
        tpose_block(tail_v, _VOCAB - _TALIGNED, _TALIGNED)


@jax.jit
def _transpose_sc(table_t, tail_t):
    mesh = plsc.VectorSubcoreMesh(core_axis_name="c", subcore_axis_name="s")
    run = functools.partial(
        pl.kernel,
        mesh=mesh,
        out_type=jax.ShapeDtypeStruct((_VOCAB, EMBED_DIM), jnp.float32),
        scratch_types=[
            pltpu.VMEM((EMBED_DIM, 128), jnp.float32),
            pltpu.VMEM((128, EMBED_DIM), jnp.float32),
            pltpu.VMEM((EMBED_DIM, _VOCAB - _TALIGNED), jnp.float32),
        ],
        compiler_params=pltpu.CompilerParams(needs_layout_passes=False),
    )(_transpose_kernel)
    return run(table_t, tail_t)


def kernel(in_idxs, embed_in):
    idx_r = in_idxs.astype(jnp.int32).reshape(_NW, _BPW // 128, 128)
    table_t = embed_in.T
    table_rm = _transpose_sc(table_t, table_t[:, _TALIGNED:])
    return _embed_gather(idx_r, table_rm)


# speedup vs baseline: 5.0679x; 5.0679x over previous
import functools

import jax
import jax.numpy as jnp
from jax import lax
from jax.experimental import pallas as pl
from jax.experimental.pallas import tpu as pltpu
from jax.experimental.pallas import tpu_sc as plsc

BATCH = 16384
EMBED_DIM = 64

_NC = 2
_NS = 16
_NW = _NC * _NS
_BPW = BATCH // _NW
_G = 16
_NGRP = _BPW // _G


def _gather_kernel(idx_hbm, table_hbm, out_hbm, idx_v, buf_v, out_v,
                   sem_a, sem_b):
    wid = lax.axis_index("s") * _NC + lax.axis_index("c")
    base = wid * _BPW
    iota = lax.iota(jnp.int32, 16)

    pltpu.sync_copy(idx_hbm.at[wid], idx_v)

    def chunk_of(g):
        return idx_v[g >> 3, pl.ds((g & 7) * 16, 16)]

    def scalar_at(vec, i):
        return jnp.max(jnp.where(iota == i, vec, 0))

    def issue_group(g, p, sem):
        chunk = chunk_of(g)
        for i in range(_G):
            off = pl.multiple_of((scalar_at(chunk, i) >> 3) * 8, 8)
            pltpu.async_copy(
                table_hbm.at[pl.ds(off, 8)], buf_v.at[p, i], sem)

    def drain_group(p, sem):
        for i in range(_G):
            pltpu.make_async_copy(
                table_hbm.at[pl.ds(0, 8)], buf_v.at[p, i], sem).wait()

    def extract_group(g, p):
        j_vec = jnp.full((16,), g >> 3, jnp.int32)
        for i in range(_G):
            r = g * _G + i
            col = jnp.full((16,), (g & 7) * 16 + i, jnp.int32)
            s_vec = plsc.load_gather(idx_v, [j_vec, col]) & 7
            for k in range(EMBED_DIM // 16):
                val = plsc.load_gather(
                    buf_v.at[p, i], [s_vec, iota + 16 * k])
                out_v[r, pl.ds(16 * k, 16)] = val

    issue_group(0, 0, sem_a)
    issue_group(1, 1, sem_b)

    def body(gp, carry):
        g0 = gp * 2
        drain_group(0, sem_a)
        extract_group(g0, 0)

        @pl.when(g0 + 2 < _NGRP)
        def _():
            issue_group(g0 + 2, 0, sem_a)

        drain_group(1, sem_b)
        extract_group(g0 + 1, 1)

        @pl.when(g0 + 3 < _NGRP)
        def _():
            issue_group(g0 + 3, 1, sem_b)

        return carry

    lax.fori_loop(0, _NGRP // 2, body, 0)

    pltpu.sync_copy(out_v, out_hbm.at[pl.ds(base, _BPW)])


@jax.jit
def _embed_gather(idx_r, table):
    mesh = plsc.VectorSubcoreMesh(core_axis_name="c", subcore_axis_name="s")
    run = functools.partial(
        pl.kernel,
        mesh=mesh,
        out_type=jax.ShapeDtypeStruct((BATCH, EMBED_DIM), jnp.float32),
        scratch_types=[
            pltpu.VMEM((_BPW // 128, 128), jnp.int32),
            pltpu.VMEM((2, _G, 8, EMBED_DIM), jnp.float32),
            pltpu.VMEM((_BPW, EMBED_DIM), jnp.float32),
            pltpu.SemaphoreType.DMA,
            pltpu.SemaphoreType.DMA,
        ],
        compiler_params=pltpu.CompilerParams(needs_layout_passes=False),
    )(_gather_kernel)
    return run(idx_r, table)


def kernel(in_idxs, embed_in):
    idx_r = in_idxs.astype(jnp.int32).reshape(_NW, _BPW // 128, 128)
    return _embed_gather(idx_r, embed_in)
